# baseline (speedup 1.0000x reference)
import functools

import jax
import jax.numpy as jnp
from jax import lax
from jax.experimental import pallas as pl
from jax.experimental.pallas import tpu as pltpu
from jax.experimental.pallas import tpu_sc as plsc

_TOP_K = 2
_SCALE = 2.5
_NUM_EXPERTS = 8
_BLOCK_T = 1024


def _sc_stream_probe(hs):
    n, h = hs.shape
    rows_per_worker = 64
    n_off = n - 32 * rows_per_worker
    mesh = plsc.VectorSubcoreMesh(core_axis_name="c", subcore_axis_name="s")

    @functools.partial(
        pl.kernel,
        out_type=jax.ShapeDtypeStruct((32, 16), jnp.float32),
        mesh=mesh,
        scratch_types=[pltpu.VMEM((16, h), jnp.float32)],
    )
    def body(hs_hbm, out_hbm, buf):
        wid = lax.axis_index("s") * 2 + lax.axis_index("c")
        base = n_off + wid * rows_per_worker
        for r in range(rows_per_worker // 16):
            pltpu.sync_copy(hs_hbm.at[pl.ds(base + r * 16, 16), :], buf)
        pltpu.sync_copy(buf.at[0, pl.ds(0, 16)], out_hbm.at[wid])

    return body(hs)


def _gate_kernel(hs_ref, w_ref, idx_ref, wt_ref):
    hs = hs_ref[...]
    w8 = w_ref[...]
    logits = lax.dot_general(
        w8, hs, (((1,), (1,)), ((), ())),
        preferred_element_type=jnp.float32,
    )
    e = lax.broadcasted_iota(jnp.int32, logits.shape, 0)
    m1 = jnp.max(logits, axis=0, keepdims=True)
    i1 = jnp.min(jnp.where(logits == m1, e, _NUM_EXPERTS), axis=0, keepdims=True)
    masked = jnp.where(e == i1, -jnp.inf, logits)
    m2 = jnp.max(masked, axis=0, keepdims=True)
    i2 = jnp.min(jnp.where(masked == m2, e, _NUM_EXPERTS), axis=0, keepdims=True)
    s1 = jax.nn.sigmoid(m1)
    s2 = jax.nn.sigmoid(m2)
    denom = s1 + s2 + 1e-20
    idx_ref[...] = jnp.concatenate([i1, i2], axis=0)
    wt_ref[...] = jnp.concatenate([s1, s2], axis=0) * (_SCALE / denom)


def kernel(hidden_states, weight):
    bsz, seq_len, h = hidden_states.shape
    n = bsz * seq_len
    hs = hidden_states.reshape(n, h).astype(jnp.float32)
    w8 = weight.astype(jnp.float32)
    grid = (n // _BLOCK_T,)
    idx_t, w_t = pl.pallas_call(
        _gate_kernel,
        grid=grid,
        in_specs=[
            pl.BlockSpec((_BLOCK_T, h), lambda i: (i, 0)),
            pl.BlockSpec((_NUM_EXPERTS, h), lambda i: (0, 0)),
        ],
        out_specs=[
            pl.BlockSpec((_TOP_K, _BLOCK_T), lambda i: (0, i)),
            pl.BlockSpec((_TOP_K, _BLOCK_T), lambda i: (0, i)),
        ],
        out_shape=[
            jax.ShapeDtypeStruct((_TOP_K, n), jnp.int32),
            jax.ShapeDtypeStruct((_TOP_K, n), jnp.float32),
        ],
        compiler_params=pltpu.CompilerParams(
            dimension_semantics=("parallel",),
        ),
    )(hs, w8)
    dummy = _sc_stream_probe(hs)
    return idx_t.T, (w_t + 0.0 * dummy[0, 0]).T


# speedup vs baseline: 1.3006x; 1.3006x over previous
import jax
import jax.numpy as jnp
from jax import lax
from jax.experimental import pallas as pl
from jax.experimental.pallas import tpu as pltpu

_TOP_K = 2
_SCALE = 2.5
_NUM_EXPERTS = 8
_BLOCK_T = 1024


def _gate_kernel(hs_ref, w_ref, idx_ref, wt_ref):
    hs = hs_ref[...]
    w8 = w_ref[...]
    logits = lax.dot_general(
        w8, hs, (((1,), (1,)), ((), ())),
        preferred_element_type=jnp.float32,
    )
    e = lax.broadcasted_iota(jnp.int32, logits.shape, 0)
    m1 = jnp.max(logits, axis=0, keepdims=True)
    i1 = jnp.min(jnp.where(logits == m1, e, _NUM_EXPERTS), axis=0, keepdims=True)
    masked = jnp.where(e == i1, -jnp.inf, logits)
    m2 = jnp.max(masked, axis=0, keepdims=True)
    i2 = jnp.min(jnp.where(masked == m2, e, _NUM_EXPERTS), axis=0, keepdims=True)
    s1 = jax.nn.sigmoid(m1)
    s2 = jax.nn.sigmoid(m2)
    denom = s1 + s2 + 1e-20
    idx_ref[...] = jnp.concatenate([i1, i2], axis=0)
    wt_ref[...] = jnp.concatenate([s1, s2], axis=0) * (_SCALE / denom)


def kernel(hidden_states, weight):
    bsz, seq_len, h = hidden_states.shape
    n = bsz * seq_len
    hs = hidden_states.reshape(n, h).astype(jnp.float32)
    w8 = weight.astype(jnp.float32)
    grid = (n // _BLOCK_T,)
    idx_t, w_t = pl.pallas_call(
        _gate_kernel,
        grid=grid,
        in_specs=[
            pl.BlockSpec((_BLOCK_T, h), lambda i: (i, 0)),
            pl.BlockSpec((_NUM_EXPERTS, h), lambda i: (0, 0)),
        ],
        out_specs=[
            pl.BlockSpec((_TOP_K, _BLOCK_T), lambda i: (0, i)),
            pl.BlockSpec((_TOP_K, _BLOCK_T), lambda i: (0, i)),
        ],
        out_shape=[
            jax.ShapeDtypeStruct((_TOP_K, n), jnp.int32),
            jax.ShapeDtypeStruct((_TOP_K, n), jnp.float32),
        ],
        compiler_params=pltpu.CompilerParams(
            dimension_semantics=("parallel",),
        ),
    )(hs, w8)
    return idx_t.T, w_t.T
